# baseline (speedup 1.0000x reference)
import functools

import jax
import jax.numpy as jnp
from jax import lax
from jax.experimental import pallas as pl
from jax.experimental.pallas import tpu as pltpu
from jax.experimental.pallas import tpu_sc as plsc

_L = 16
_CHUNK = 128


@functools.lru_cache(maxsize=None)
def _build(seq_len: int, vocab: int, dim: int):
    info = plsc.get_sparse_core_info()
    nc, ns = info.num_cores, info.num_subcores
    nw = nc * ns
    assert seq_len % (nw * _CHUNK) == 0 or seq_len % nw == 0
    bpw = seq_len // nw
    nchunk = max(1, bpw // _CHUNK)
    chunk = bpw // nchunk
    assert dim % _L == 0

    mesh = plsc.VectorSubcoreMesh(core_axis_name="c", subcore_axis_name="s")

    @functools.partial(
        pl.kernel,
        mesh=mesh,
        out_type=jax.ShapeDtypeStruct((seq_len, dim), jnp.float32),
        scratch_types=[
            pltpu.VMEM((nchunk, chunk), jnp.int32),
            pltpu.VMEM((bpw, dim), jnp.float32),
            pltpu.VMEM((bpw, dim), jnp.float32),
            pltpu.SemaphoreType.DMA,
        ],
        compiler_params=pltpu.CompilerParams(use_tc_tiling_on_sc=False),
    )
    def emb(x_hbm, wt_hbm, pt_hbm, out_hbm, idx_v, rows_v, pos_v, sem):
        wid = lax.axis_index("s") * nc + lax.axis_index("c")
        base = wid * bpw

        for j in range(nchunk):
            pltpu.sync_copy(x_hbm.at[pl.ds(base + j * chunk, chunk)], idx_v.at[j])

        cps = [
            pltpu.async_copy(
                wt_hbm.at[idx_v.at[j]],
                rows_v.at[pl.ds(j * chunk, chunk)],
                sem,
            )
            for j in range(nchunk)
        ]

        pltpu.sync_copy(pt_hbm.at[pl.ds(base, bpw)], pos_v)

        for cp in cps:
            cp.wait()

        nvec = dim // _L

        def add_row(i):
            for d in range(nvec):
                sl = pl.ds(d * _L, _L)
                rows_v[i, sl] = rows_v[i, sl] + pos_v[i, sl]

        pl.loop(0, bpw)(add_row)

        pltpu.sync_copy(rows_v, out_hbm.at[pl.ds(base, bpw)])

    return emb


def kernel(x, word_table, pos_table):
    seq_len = x.shape[0]
    vocab, dim = word_table.shape
    emb = _build(seq_len, vocab, dim)
    return emb(x.astype(jnp.int32), word_table, pos_table)


# speedup vs baseline: 4.2795x; 4.2795x over previous
import functools

import jax
import jax.numpy as jnp
from jax import lax
from jax.experimental import pallas as pl
from jax.experimental.pallas import tpu as pltpu
from jax.experimental.pallas import tpu_sc as plsc

_L = 16
_TILE = 128
_NBUF = 4


@functools.lru_cache(maxsize=None)
def _build(seq_len: int, vocab: int, dim: int):
    info = plsc.get_sparse_core_info()
    nc, ns = info.num_cores, info.num_subcores
    nw = nc * ns
    assert seq_len % (nw * _L) == 0
    bpw = seq_len // nw
    ngroups = bpw // _L
    assert dim % _L == 0
    nr = dim // _L

    mesh = plsc.VectorSubcoreMesh(core_axis_name="c", subcore_axis_name="s")

    @functools.partial(
        pl.kernel,
        mesh=mesh,
        out_type=jax.ShapeDtypeStruct((dim, seq_len), jnp.float32),
        scratch_types=[
            pltpu.VMEM((bpw,), jnp.int32),
            pltpu.VMEM((_NBUF * dim, _TILE), jnp.float32),
            pltpu.VMEM((dim, bpw), jnp.float32),
            pltpu.VMEM((dim, bpw), jnp.float32),
            [pltpu.SemaphoreType.DMA] * _NBUF,
            pltpu.SemaphoreType.DMA,
        ],
        compiler_params=pltpu.CompilerParams(needs_layout_passes=False),
    )
    def emb(x_hbm, wt_hbm, pt_hbm, out_hbm, idx_v, ring_v, buf_v, pos_v, sems, psem):
        wid = lax.axis_index("s") * nc + lax.axis_index("c")
        base = wid * bpw

        pltpu.sync_copy(x_hbm.at[pl.ds(base, bpw)], idx_v)
        pltpu.async_copy(pt_hbm.at[:, pl.ds(base, bpw)], pos_v, psem).wait()

        iota = lax.iota(jnp.int32, _L)

        def fire(k, t128):
            tk = pl.multiple_of(t128[k], _TILE)
            b = k % _NBUF
            pltpu.async_copy(
                wt_hbm.at[:, pl.ds(tk, _TILE)],
                ring_v.at[pl.ds(b * dim, dim), :],
                sems[b],
            )

        def group(g):
            vec = idx_v[pl.ds(g * _L, _L)]
            t128 = vec & jnp.int32(-_TILE)
            lanes = vec & jnp.int32(_TILE - 1)
            for k in range(_NBUF):
                fire(k, t128)
            for k in range(_L):
                b = k % _NBUF
                pltpu.make_async_copy(
                    wt_hbm.at[:, pl.ds(0, _TILE)],
                    ring_v.at[pl.ds(b * dim, dim), :],
                    sems[b],
                ).wait()
                col = jnp.full((_L,), g * _L + k, jnp.int32)
                lane = jnp.full((_L,), lanes[k], jnp.int32)
                for r in range(nr):
                    rows = iota + (b * dim + r * _L)
                    val = plsc.load_gather(ring_v, [rows, lane])
                    prow = iota + r * _L
                    pv = plsc.load_gather(pos_v, [prow, col])
                    plsc.store_scatter(buf_v, [prow, col], val + pv)
                if k + _NBUF < _L:
                    fire(k + _NBUF, t128)

        pl.loop(0, ngroups)(group)

        pltpu.sync_copy(buf_v, out_hbm.at[:, pl.ds(base, bpw)])

    return emb


def kernel(x, word_table, pos_table):
    seq_len = x.shape[0]
    vocab, dim = word_table.shape
    emb = _build(seq_len, vocab, dim)
    out_t = emb(x.astype(jnp.int32), word_table.T, pos_table[:seq_len].T)
    return out_t.T


# speedup vs baseline: 4.7310x; 1.1055x over previous
import functools

import jax
import jax.numpy as jnp
from jax import lax
from jax.experimental import pallas as pl
from jax.experimental.pallas import tpu as pltpu
from jax.experimental.pallas import tpu_sc as plsc

_L = 16
_TILE = 128
_NBUF = 8


@functools.lru_cache(maxsize=None)
def _build(seq_len: int, vocab: int, dim: int):
    info = plsc.get_sparse_core_info()
    nc, ns = info.num_cores, info.num_subcores
    nw = nc * ns
    assert seq_len % (nw * _L) == 0
    bpw = seq_len // nw
    ngroups = bpw // _L
    assert dim % _L == 0
    nr = dim // _L

    mesh = plsc.VectorSubcoreMesh(core_axis_name="c", subcore_axis_name="s")

    @functools.partial(
        pl.kernel,
        mesh=mesh,
        out_type=jax.ShapeDtypeStruct((dim, seq_len), jnp.float32),
        scratch_types=[
            pltpu.VMEM((bpw,), jnp.int32),
            pltpu.VMEM((_NBUF * dim, _TILE), jnp.float32),
            pltpu.VMEM((dim, bpw), jnp.float32),
            pltpu.VMEM((dim, bpw), jnp.float32),
            [pltpu.SemaphoreType.DMA] * _NBUF,
            pltpu.SemaphoreType.DMA,
        ],
        compiler_params=pltpu.CompilerParams(needs_layout_passes=False),
    )
    def emb(x_hbm, wt_hbm, pt_hbm, out_hbm, idx_v, ring_v, buf_v, pos_v, sems, psem):
        wid = lax.axis_index("s") * nc + lax.axis_index("c")
        base = wid * bpw

        pltpu.sync_copy(x_hbm.at[pl.ds(base, bpw)], idx_v)
        pltpu.async_copy(pt_hbm.at[:, pl.ds(base, bpw)], pos_v, psem).wait()

        iota = lax.iota(jnp.int32, _L)

        def fire(k, t128):
            tk = pl.multiple_of(t128[k], _TILE)
            b = k % _NBUF
            pltpu.async_copy(
                wt_hbm.at[:, pl.ds(tk, _TILE)],
                ring_v.at[pl.ds(b * dim, dim), :],
                sems[b],
            )

        def group(g):
            vec = idx_v[pl.ds(g * _L, _L)]
            t128 = vec & jnp.int32(-_TILE)
            lanes = vec & jnp.int32(_TILE - 1)
            for k in range(_NBUF):
                fire(k, t128)
            for k in range(_L):
                b = k % _NBUF
                pltpu.make_async_copy(
                    wt_hbm.at[:, pl.ds(0, _TILE)],
                    ring_v.at[pl.ds(b * dim, dim), :],
                    sems[b],
                ).wait()
                col = jnp.full((_L,), g * _L + k, jnp.int32)
                lane = jnp.full((_L,), lanes[k], jnp.int32)
                for r in range(nr):
                    rows = iota + (b * dim + r * _L)
                    val = plsc.load_gather(ring_v, [rows, lane])
                    prow = iota + r * _L
                    pv = plsc.load_gather(pos_v, [prow, col])
                    plsc.store_scatter(buf_v, [prow, col], val + pv)
                if k + _NBUF < _L:
                    fire(k + _NBUF, t128)

        pl.loop(0, ngroups)(group)

        pltpu.sync_copy(buf_v, out_hbm.at[:, pl.ds(base, bpw)])

    return emb


def kernel(x, word_table, pos_table):
    seq_len = x.shape[0]
    vocab, dim = word_table.shape
    emb = _build(seq_len, vocab, dim)
    out_t = emb(x.astype(jnp.int32), word_table.T, pos_table[:seq_len].T)
    return out_t.T
